# baseline (speedup 1.0000x reference)
import functools

import jax
import jax.numpy as jnp
from jax import lax
from jax.experimental import pallas as pl
from jax.experimental.pallas import tpu as pltpu
from jax.experimental.pallas import tpu_sc as plsc

_N = 50000
_K = 16
_E = _N * _K
_T = 400
_NC = 2
_NS = 16
_CH = 1000


def _sc_gather6(tables, cols):
    e = cols.shape[0]
    nw = _NC * _NS
    b_per_w = e // nw
    n_ch = b_per_w // _CH
    mesh = plsc.VectorSubcoreMesh(core_axis_name="c", subcore_axis_name="s")

    @functools.partial(
        pl.kernel,
        out_type=[jax.ShapeDtypeStruct((e,), jnp.float32) for _ in range(6)],
        mesh=mesh,
        scratch_types=[pltpu.VMEM((_CH,), jnp.int32)]
        + [pltpu.VMEM((_CH,), jnp.float32) for _ in range(6)]
        + [pltpu.SemaphoreType.DMA],
    )
    def gather_kernel(*refs):
        tbls = refs[0:6]
        idx_hbm = refs[6]
        outs = refs[7:13]
        idx_v = refs[13]
        vals = refs[14:20]
        sem = refs[20]
        wid = lax.axis_index("s") * _NC + lax.axis_index("c")
        base = wid * b_per_w

        @pl.loop(0, n_ch)
        def _(c):
            off = base + c * _CH
            pltpu.sync_copy(idx_hbm.at[pl.ds(off, _CH)], idx_v)
            copies = [
                pltpu.async_copy(tbls[j].at[idx_v], vals[j], sem)
                for j in range(6)
            ]
            for cp in copies:
                cp.wait()
            for j in range(6):
                pltpu.sync_copy(vals[j], outs[j].at[pl.ds(off, _CH)])

    return gather_kernel(*tables, cols)


def _tc_body(std_ref, pos_ref, nrm_ref, g0, g1, g2, g3, g4, g5, w_ref, *rest):
    (rmat,
     p1, b1t, bd12, b2t1,
     gw1, gb1, gw2, gb2,
     bda2, tb2, b1t2, bd22, b2t2,
     g2w1, g2b1, g2w2, g2b2,
     bda3, tb3, b1t3, bd23, b2t3,
     g3w1, g3b1, g3w2, g3b2,
     bd4a, tb4, pc4, b4t, bd4b, b4b,
     out_ref) = rest

    def mm(a, b):
        return jnp.dot(a, b[...], preferred_element_type=jnp.float32)

    s = 0.2 / std_ref[0, 0]
    prx, pry, prz = pos_ref[:, 0:1], pos_ref[:, 1:2], pos_ref[:, 2:3]
    nrx, nry, nrz = nrm_ref[:, 0:1], nrm_ref[:, 1:2], nrm_ref[:, 2:3]
    pcx, pcy, pcz = g0[...], g1[...], g2[...]
    ncx, ncy, ncz = g3[...], g4[...], g5[...]

    cx = (pcx - prx) * s
    cy = (pcy - pry) * s
    cz = (pcz - prz) * s

    def sqn(u0, u1, u2):
        return u0 * u0 + u1 * u1 + u2 * u2

    s1 = sqn(nry * cz - nrz * cy, nrz * cx - nrx * cz, nrx * cy - nry * cx)
    s2 = sqn(ncy * cz - ncz * cy, ncz * cx - ncx * cz, ncx * cy - ncy * cx)
    s3 = sqn(nry * ncz - nrz * ncy, nrz * ncx - nrx * ncz, nrx * ncy - nry * ncx)
    rt = jnp.sqrt(jnp.concatenate([s1, s2, s3, sqn(cx, cy, cz)], axis=1))
    dots = jnp.concatenate(
        [nrx * cx + nry * cy + nrz * cz,
         ncx * cx + ncy * cy + ncz * cz,
         nrx * ncx + nry * ncy + nrz * ncz], axis=1)
    ang = jnp.arctan2(rt[:, 0:48], dots)

    x128 = jnp.concatenate([cx, cy, cz, w_ref[...], rt[:, 48:64], ang], axis=1)
    h = jnp.maximum(mm(x128, p1) + b1t[...], 0.0)
    x16 = mm(h, bd12) + b2t1[...]

    gx = mm(x16, rmat)
    gin = jnp.concatenate([gx, nrm_ref[...]], axis=1)
    hg = jnp.maximum(mm(gin, gw1) + gb1[...], 0.0)
    xg = mm(hg, gw2) + gb2[...]

    h = jnp.maximum(mm(x16, bda2) + mm(xg, tb2) + b1t2[...], 0.0)
    x16 = mm(h, bd22) + b2t2[...]

    gx = mm(x16, rmat)
    hg = jnp.maximum(mm(gx, g2w1) + g2b1[...], 0.0)
    xg = mm(hg, g2w2) + g2b2[...]

    h = jnp.maximum(mm(x16, bda3) + mm(xg, tb3) + b1t3[...], 0.0)
    x16 = mm(h, bd23) + b2t3[...]

    gx = mm(x16, rmat)
    hg = jnp.maximum(mm(gx, g3w1) + g3b1[...], 0.0)
    xg = mm(hg, g3w2) + g3b2[...]

    qw, qx, qy, qz = xg[:, 0:1], xg[:, 1:2], xg[:, 2:3], xg[:, 3:4]
    d = qw * qw + qx * qx + qy * qy + qz * qz + 1e-16
    r = 1.0 / d
    m00 = d - 2 * (qy * qy + qz * qz)
    m01 = 2 * (qx * qy - qw * qz)
    m02 = 2 * (qx * qz + qw * qy)
    m10 = 2 * (qx * qy + qw * qz)
    m11 = d - 2 * (qx * qx + qz * qz)
    m12 = 2 * (qy * qz - qw * qx)
    m20 = 2 * (qx * qz - qw * qy)
    m21 = 2 * (qy * qz + qw * qx)
    m22 = d - 2 * (qx * qx + qy * qy)
    rcx = (m00 * cx + m01 * cy + m02 * cz) * r
    rcy = (m10 * cx + m11 * cy + m12 * cz) * r
    rcz = (m20 * cx + m21 * cy + m22 * cz) * r
    rc = jnp.concatenate([rcx, rcy, rcz], axis=1)

    h = jnp.maximum(
        mm(x16, bd4a) + mm(xg[:, 4:12], tb4) + mm(rc, pc4) + b4t[...], 0.0
    )
    y = mm(h, bd4b) + b4b[...]

    ymax = jnp.max(y, axis=1, keepdims=True)
    ey = jnp.exp(y - ymax)
    out_ref[...] = ey / jnp.sum(ey, axis=1, keepdims=True)


def _make_consts(params):
    eye = jnp.eye(_K, dtype=jnp.float32)

    def bd(w):
        return jnp.kron(eye, w)

    def fold_first(w, fin):
        return jnp.einsum("fh,kK->fkKh", w, eye).reshape(fin * _K, _K * w.shape[1])

    def tile_b(b):
        return jnp.tile(b.reshape(1, -1), (1, _K))

    l1w1, l1b1, l1w2, l1b2 = params["layer1"]
    gw1, gb1, gw2, gb2 = params["layerg"]
    l2w1, l2b1, l2w2, l2b2 = params["layer2"]
    g2w1, g2b1, g2w2, g2b2 = params["layerg2"]
    l3w1, l3b1, l3w2, l3b2 = params["layer3"]
    g3w1, g3b1, g3w2, g3b2 = params["layerg3"]
    l4w1, l4b1, l4w2, l4b2 = params["layer4"]

    consts = [
        jnp.tile(eye, (_K, 1)) / _K,
        fold_first(l1w1, 8), tile_b(l1b1), bd(l1w2), tile_b(l1b2),
        gw1, gb1.reshape(1, -1), gw2, gb2.reshape(1, -1),
        bd(l2w1[:16]), jnp.tile(l2w1[16:24], (1, _K)), tile_b(l2b1),
        bd(l2w2), tile_b(l2b2),
        g2w1, g2b1.reshape(1, -1), g2w2, g2b2.reshape(1, -1),
        bd(l3w1[:16]), jnp.tile(l3w1[16:24], (1, _K)), tile_b(l3b1),
        bd(l3w2), tile_b(l3b2),
        g3w1, g3b1.reshape(1, -1), g3w2, g3b2.reshape(1, -1),
        bd(l4w1[:16]), jnp.tile(l4w1[16:24], (1, _K)), fold_first(l4w1[24:27], 3),
        tile_b(l4b1), bd(l4w2), tile_b(l4b2),
    ]
    return consts


def _tc_forward(std, pos, nrm, g6, w, consts):
    n = pos.shape[0]
    nblk = n // _T
    in_specs = [
        pl.BlockSpec(memory_space=pltpu.SMEM),
        pl.BlockSpec((_T, 3), lambda i: (i, 0)),
        pl.BlockSpec((_T, 3), lambda i: (i, 0)),
    ] + [pl.BlockSpec((_T, _K), lambda i: (i, 0)) for _ in range(7)] + [
        pl.BlockSpec(c.shape, lambda i: tuple([0] * c.ndim)) for c in consts
    ]
    out = pl.pallas_call(
        _tc_body,
        grid=(nblk,),
        in_specs=in_specs,
        out_specs=pl.BlockSpec((_T, _K), lambda i: (i, 0)),
        out_shape=jax.ShapeDtypeStruct((n, _K), jnp.float32),
        compiler_params=pltpu.CompilerParams(
            dimension_semantics=("parallel",)
        ),
    )(std, pos, nrm, *g6, w, *consts)
    return out


_S = 25


def kernel(pos, old_weights, normals, edge_index, dense_l, stddev, params):
    cols = edge_index[1]
    tables = [pos[:, 0], pos[:, 1], pos[:, 2],
              normals[:, 0], normals[:, 1], normals[:, 2]]
    w = old_weights.reshape(_N, _K)
    std = stddev.reshape(1, 1)
    consts = _make_consts(params)

    nn = _N // _S
    ne = _E // _S
    gathered = [
        _sc_gather6(tables, lax.dynamic_slice_in_dim(cols, c * ne, ne))
        for c in range(_S)
    ]
    outs = []
    for c in range(_S):
        g6 = [a.reshape(nn, _K) for a in gathered[c]]
        outs.append(
            _tc_forward(
                std,
                lax.dynamic_slice_in_dim(pos, c * nn, nn),
                lax.dynamic_slice_in_dim(normals, c * nn, nn),
                g6,
                lax.dynamic_slice_in_dim(w, c * nn, nn),
                consts,
            )
        )
    return jnp.concatenate(outs, axis=0)


# speedup vs baseline: 1.3414x; 1.3414x over previous
import functools

import jax
import jax.numpy as jnp
from jax import lax
from jax.experimental import pallas as pl
from jax.experimental.pallas import tpu as pltpu
from jax.experimental.pallas import tpu_sc as plsc

_N = 50000
_K = 16
_E = _N * _K
_T = 400
_NC = 2
_NS = 16
_CH = 5000


def _sc_gather6(tables, cols, ebase, e):
    nw = _NC * _NS
    b_per_w = e // nw
    n_ch = b_per_w // _CH
    mesh = plsc.VectorSubcoreMesh(core_axis_name="c", subcore_axis_name="s")

    @functools.partial(
        pl.kernel,
        out_type=[jax.ShapeDtypeStruct((e,), jnp.float32) for _ in range(6)],
        mesh=mesh,
        scratch_types=[pltpu.VMEM((_CH,), jnp.int32)]
        + [pltpu.VMEM((_CH,), jnp.float32) for _ in range(6)]
        + [pltpu.SemaphoreType.DMA],
    )
    def gather_kernel(*refs):
        tbls = refs[0:6]
        idx_hbm = refs[6]
        outs = refs[7:13]
        idx_v = refs[13]
        vals = refs[14:20]
        sem = refs[20]
        wid = lax.axis_index("s") * _NC + lax.axis_index("c")
        base = wid * b_per_w

        @pl.loop(0, n_ch)
        def _(c):
            off = base + c * _CH
            pltpu.sync_copy(idx_hbm.at[pl.ds(ebase + off, _CH)], idx_v)
            copies = [
                pltpu.async_copy(tbls[j].at[idx_v], vals[j], sem)
                for j in range(6)
            ]
            for cp in copies:
                cp.wait()
            for j in range(6):
                pltpu.sync_copy(vals[j], outs[j].at[pl.ds(off, _CH)])

    return gather_kernel(*tables, cols)


def _tc_body(std_ref, pos_ref, nrm_ref, g0, g1, g2, g3, g4, g5, w_ref, *rest):
    (rmat,
     p1, b1t, bd12, b2t1,
     gw1, gb1, gw2, gb2,
     bda2, tb2, b1t2, bd22, b2t2,
     g2w1, g2b1, g2w2, g2b2,
     bda3, tb3, b1t3, bd23, b2t3,
     g3w1, g3b1, g3w2, g3b2,
     bd4a, tb4, pc4, b4t, bd4b, b4b,
     out_ref) = rest

    def mm(a, b):
        return jnp.dot(a, b[...], preferred_element_type=jnp.float32)

    s = 0.2 / std_ref[0, 0]
    prx, pry, prz = pos_ref[:, 0:1], pos_ref[:, 1:2], pos_ref[:, 2:3]
    nrx, nry, nrz = nrm_ref[:, 0:1], nrm_ref[:, 1:2], nrm_ref[:, 2:3]
    pcx, pcy, pcz = g0[...], g1[...], g2[...]
    ncx, ncy, ncz = g3[...], g4[...], g5[...]

    cx = (pcx - prx) * s
    cy = (pcy - pry) * s
    cz = (pcz - prz) * s

    def sqn(u0, u1, u2):
        return u0 * u0 + u1 * u1 + u2 * u2

    s1 = sqn(nry * cz - nrz * cy, nrz * cx - nrx * cz, nrx * cy - nry * cx)
    s2 = sqn(ncy * cz - ncz * cy, ncz * cx - ncx * cz, ncx * cy - ncy * cx)
    s3 = sqn(nry * ncz - nrz * ncy, nrz * ncx - nrx * ncz, nrx * ncy - nry * ncx)
    rt = jnp.sqrt(jnp.concatenate([s1, s2, s3, sqn(cx, cy, cz)], axis=1))
    dots = jnp.concatenate(
        [nrx * cx + nry * cy + nrz * cz,
         ncx * cx + ncy * cy + ncz * cz,
         nrx * ncx + nry * ncy + nrz * ncz], axis=1)
    ang = jnp.arctan2(rt[:, 0:48], dots)

    x128 = jnp.concatenate([cx, cy, cz, w_ref[...], rt[:, 48:64], ang], axis=1)
    h = jnp.maximum(mm(x128, p1) + b1t[...], 0.0)
    x16 = mm(h, bd12) + b2t1[...]

    gx = mm(x16, rmat)
    gin = jnp.concatenate([gx, nrm_ref[...]], axis=1)
    hg = jnp.maximum(mm(gin, gw1) + gb1[...], 0.0)
    xg = mm(hg, gw2) + gb2[...]

    h = jnp.maximum(mm(x16, bda2) + mm(xg, tb2) + b1t2[...], 0.0)
    x16 = mm(h, bd22) + b2t2[...]

    gx = mm(x16, rmat)
    hg = jnp.maximum(mm(gx, g2w1) + g2b1[...], 0.0)
    xg = mm(hg, g2w2) + g2b2[...]

    h = jnp.maximum(mm(x16, bda3) + mm(xg, tb3) + b1t3[...], 0.0)
    x16 = mm(h, bd23) + b2t3[...]

    gx = mm(x16, rmat)
    hg = jnp.maximum(mm(gx, g3w1) + g3b1[...], 0.0)
    xg = mm(hg, g3w2) + g3b2[...]

    qw, qx, qy, qz = xg[:, 0:1], xg[:, 1:2], xg[:, 2:3], xg[:, 3:4]
    d = qw * qw + qx * qx + qy * qy + qz * qz + 1e-16
    r = 1.0 / d
    m00 = d - 2 * (qy * qy + qz * qz)
    m01 = 2 * (qx * qy - qw * qz)
    m02 = 2 * (qx * qz + qw * qy)
    m10 = 2 * (qx * qy + qw * qz)
    m11 = d - 2 * (qx * qx + qz * qz)
    m12 = 2 * (qy * qz - qw * qx)
    m20 = 2 * (qx * qz - qw * qy)
    m21 = 2 * (qy * qz + qw * qx)
    m22 = d - 2 * (qx * qx + qy * qy)
    rcx = (m00 * cx + m01 * cy + m02 * cz) * r
    rcy = (m10 * cx + m11 * cy + m12 * cz) * r
    rcz = (m20 * cx + m21 * cy + m22 * cz) * r
    rc = jnp.concatenate([rcx, rcy, rcz], axis=1)

    h = jnp.maximum(
        mm(x16, bd4a) + mm(xg[:, 4:12], tb4) + mm(rc, pc4) + b4t[...], 0.0
    )
    y = mm(h, bd4b) + b4b[...]

    ymax = jnp.max(y, axis=1, keepdims=True)
    ey = jnp.exp(y - ymax)
    out_ref[...] = ey / jnp.sum(ey, axis=1, keepdims=True)


def _make_consts(params):
    eye = jnp.eye(_K, dtype=jnp.float32)

    def bd(w):
        return jnp.kron(eye, w)

    def fold_first(w, fin):
        return jnp.einsum("fh,kK->fkKh", w, eye).reshape(fin * _K, _K * w.shape[1])

    def tile_b(b):
        return jnp.tile(b.reshape(1, -1), (1, _K))

    l1w1, l1b1, l1w2, l1b2 = params["layer1"]
    gw1, gb1, gw2, gb2 = params["layerg"]
    l2w1, l2b1, l2w2, l2b2 = params["layer2"]
    g2w1, g2b1, g2w2, g2b2 = params["layerg2"]
    l3w1, l3b1, l3w2, l3b2 = params["layer3"]
    g3w1, g3b1, g3w2, g3b2 = params["layerg3"]
    l4w1, l4b1, l4w2, l4b2 = params["layer4"]

    consts = [
        jnp.tile(eye, (_K, 1)) / _K,
        fold_first(l1w1, 8), tile_b(l1b1), bd(l1w2), tile_b(l1b2),
        gw1, gb1.reshape(1, -1), gw2, gb2.reshape(1, -1),
        bd(l2w1[:16]), jnp.tile(l2w1[16:24], (1, _K)), tile_b(l2b1),
        bd(l2w2), tile_b(l2b2),
        g2w1, g2b1.reshape(1, -1), g2w2, g2b2.reshape(1, -1),
        bd(l3w1[:16]), jnp.tile(l3w1[16:24], (1, _K)), tile_b(l3b1),
        bd(l3w2), tile_b(l3b2),
        g3w1, g3b1.reshape(1, -1), g3w2, g3b2.reshape(1, -1),
        bd(l4w1[:16]), jnp.tile(l4w1[16:24], (1, _K)), fold_first(l4w1[24:27], 3),
        tile_b(l4b1), bd(l4w2), tile_b(l4b2),
    ]
    return consts


def _tc_forward(std, pos, nrm, g6, w, consts, blk0, nblk):
    nn = nblk * _T
    in_specs = [
        pl.BlockSpec(memory_space=pltpu.SMEM),
        pl.BlockSpec((_T, 3), lambda i: (blk0 + i, 0)),
        pl.BlockSpec((_T, 3), lambda i: (blk0 + i, 0)),
    ] + [pl.BlockSpec((_T, _K), lambda i: (i, 0)) for _ in range(6)] + [
        pl.BlockSpec((_T, _K), lambda i: (blk0 + i, 0)),
    ] + [
        pl.BlockSpec(c.shape, lambda i: tuple([0] * c.ndim)) for c in consts
    ]
    out = pl.pallas_call(
        _tc_body,
        grid=(nblk,),
        in_specs=in_specs,
        out_specs=pl.BlockSpec((_T, _K), lambda i: (i, 0)),
        out_shape=jax.ShapeDtypeStruct((nn, _K), jnp.float32),
        compiler_params=pltpu.CompilerParams(
            dimension_semantics=("parallel",)
        ),
    )(std, pos, nrm, *g6, w, *consts)
    return out


_S = 5


def kernel(pos, old_weights, normals, edge_index, dense_l, stddev, params):
    cols = edge_index[1]
    tables = [pos[:, 0], pos[:, 1], pos[:, 2],
              normals[:, 0], normals[:, 1], normals[:, 2]]
    w = old_weights.reshape(_N, _K)
    std = stddev.reshape(1, 1)
    consts = _make_consts(params)

    nn = _N // _S
    ne = _E // _S
    gathered = [_sc_gather6(tables, cols, c * ne, ne) for c in range(_S)]
    outs = []
    for c in range(_S):
        g6 = [a.reshape(nn, _K) for a in gathered[c]]
        outs.append(
            _tc_forward(std, pos, normals, g6, w, consts,
                        c * (nn // _T), nn // _T)
        )
    return jnp.concatenate(outs, axis=0)


# speedup vs baseline: 1.4792x; 1.1027x over previous
import functools

import jax
import jax.numpy as jnp
from jax import lax
from jax.experimental import pallas as pl
from jax.experimental.pallas import tpu as pltpu
from jax.experimental.pallas import tpu_sc as plsc

_N = 50000
_K = 16
_E = _N * _K
_T = 1000
_NC = 2
_NS = 16
_CH = 5000


def _sc_gather6(tables, cols, ebase, e):
    nw = _NC * _NS
    b_per_w = e // nw
    n_ch = b_per_w // _CH
    mesh = plsc.VectorSubcoreMesh(core_axis_name="c", subcore_axis_name="s")

    @functools.partial(
        pl.kernel,
        out_type=[jax.ShapeDtypeStruct((e,), jnp.float32) for _ in range(6)],
        mesh=mesh,
        scratch_types=[pltpu.VMEM((_CH,), jnp.int32)]
        + [pltpu.VMEM((_CH,), jnp.float32) for _ in range(6)]
        + [pltpu.SemaphoreType.DMA],
    )
    def gather_kernel(*refs):
        tbls = refs[0:6]
        idx_hbm = refs[6]
        outs = refs[7:13]
        idx_v = refs[13]
        vals = refs[14:20]
        sem = refs[20]
        wid = lax.axis_index("s") * _NC + lax.axis_index("c")
        base = wid * b_per_w

        @pl.loop(0, n_ch)
        def _(c):
            off = base + c * _CH
            pltpu.sync_copy(idx_hbm.at[pl.ds(ebase + off, _CH)], idx_v)
            copies = [
                pltpu.async_copy(tbls[j].at[idx_v], vals[j], sem)
                for j in range(6)
            ]
            for cp in copies:
                cp.wait()
            for j in range(6):
                pltpu.sync_copy(vals[j], outs[j].at[pl.ds(off, _CH)])

    return gather_kernel(*tables, cols)


def _tc_body(std_ref, pos_ref, nrm_ref, g0, g1, g2, g3, g4, g5, w_ref, *rest):
    (rmat,
     p1, b1t, bd12, b2t1,
     gw1, gb1, gw2, gb2,
     bda2, tb2, b1t2, bd22, b2t2,
     g2w1, g2b1, g2w2, g2b2,
     bda3, tb3, b1t3, bd23, b2t3,
     g3w1, g3b1, g3w2, g3b2,
     bd4a, tb4, pc4, b4t, bd4b, b4b,
     out_ref) = rest

    def mm(a, b):
        return jnp.dot(a, b[...], preferred_element_type=jnp.float32)

    s = 0.2 / std_ref[0, 0]
    prx, pry, prz = pos_ref[:, 0:1], pos_ref[:, 1:2], pos_ref[:, 2:3]
    nrx, nry, nrz = nrm_ref[:, 0:1], nrm_ref[:, 1:2], nrm_ref[:, 2:3]

    pcx, pcy, pcz = g0[...], g1[...], g2[...]
    ncx, ncy, ncz = g3[...], g4[...], g5[...]

    cx = (pcx - prx) * s
    cy = (pcy - pry) * s
    cz = (pcz - prz) * s

    def sqn(u0, u1, u2):
        return u0 * u0 + u1 * u1 + u2 * u2

    s1 = sqn(nry * cz - nrz * cy, nrz * cx - nrx * cz, nrx * cy - nry * cx)
    s2 = sqn(ncy * cz - ncz * cy, ncz * cx - ncx * cz, ncx * cy - ncy * cx)
    s3 = sqn(nry * ncz - nrz * ncy, nrz * ncx - nrx * ncz, nrx * ncy - nry * ncx)
    rt = jnp.sqrt(jnp.concatenate([s1, s2, s3, sqn(cx, cy, cz)], axis=1))
    dots = jnp.concatenate(
        [nrx * cx + nry * cy + nrz * cz,
         ncx * cx + ncy * cy + ncz * cz,
         nrx * ncx + nry * ncy + nrz * ncz], axis=1)
    ang = jnp.arctan2(rt[:, 0:48], dots)

    x128 = jnp.concatenate([cx, cy, cz, w_ref[...], rt[:, 48:64], ang], axis=1)
    h = jnp.maximum(mm(x128, p1) + b1t[...], 0.0)
    x16 = mm(h, bd12) + b2t1[...]

    gx = mm(x16, rmat)
    gin = jnp.concatenate([gx, nrm_ref[...]], axis=1)
    hg = jnp.maximum(mm(gin, gw1) + gb1[...], 0.0)
    xg = mm(hg, gw2) + gb2[...]

    h = jnp.maximum(mm(x16, bda2) + mm(xg, tb2) + b1t2[...], 0.0)
    x16 = mm(h, bd22) + b2t2[...]

    gx = mm(x16, rmat)
    hg = jnp.maximum(mm(gx, g2w1) + g2b1[...], 0.0)
    xg = mm(hg, g2w2) + g2b2[...]

    h = jnp.maximum(mm(x16, bda3) + mm(xg, tb3) + b1t3[...], 0.0)
    x16 = mm(h, bd23) + b2t3[...]

    gx = mm(x16, rmat)
    hg = jnp.maximum(mm(gx, g3w1) + g3b1[...], 0.0)
    xg = mm(hg, g3w2) + g3b2[...]

    qw, qx, qy, qz = xg[:, 0:1], xg[:, 1:2], xg[:, 2:3], xg[:, 3:4]
    d = qw * qw + qx * qx + qy * qy + qz * qz + 1e-16
    r = 1.0 / d
    m00 = d - 2 * (qy * qy + qz * qz)
    m01 = 2 * (qx * qy - qw * qz)
    m02 = 2 * (qx * qz + qw * qy)
    m10 = 2 * (qx * qy + qw * qz)
    m11 = d - 2 * (qx * qx + qz * qz)
    m12 = 2 * (qy * qz - qw * qx)
    m20 = 2 * (qx * qz - qw * qy)
    m21 = 2 * (qy * qz + qw * qx)
    m22 = d - 2 * (qx * qx + qy * qy)
    rcx = (m00 * cx + m01 * cy + m02 * cz) * r
    rcy = (m10 * cx + m11 * cy + m12 * cz) * r
    rcz = (m20 * cx + m21 * cy + m22 * cz) * r
    rc = jnp.concatenate([rcx, rcy, rcz], axis=1)

    h = jnp.maximum(
        mm(x16, bd4a) + mm(xg[:, 4:12], tb4) + mm(rc, pc4) + b4t[...], 0.0
    )
    y = mm(h, bd4b) + b4b[...]

    ymax = jnp.max(y, axis=1, keepdims=True)
    ey = jnp.exp(y - ymax)
    out_ref[...] = ey / jnp.sum(ey, axis=1, keepdims=True)


def _make_consts(params):
    eye = jnp.eye(_K, dtype=jnp.float32)

    def bd(w):
        return jnp.kron(eye, w)

    def fold_first(w, fin):
        return jnp.einsum("fh,kK->fkKh", w, eye).reshape(fin * _K, _K * w.shape[1])

    def tile_b(b):
        return jnp.tile(b.reshape(1, -1), (1, _K))

    l1w1, l1b1, l1w2, l1b2 = params["layer1"]
    gw1, gb1, gw2, gb2 = params["layerg"]
    l2w1, l2b1, l2w2, l2b2 = params["layer2"]
    g2w1, g2b1, g2w2, g2b2 = params["layerg2"]
    l3w1, l3b1, l3w2, l3b2 = params["layer3"]
    g3w1, g3b1, g3w2, g3b2 = params["layerg3"]
    l4w1, l4b1, l4w2, l4b2 = params["layer4"]

    consts = [
        jnp.tile(eye, (_K, 1)) / _K,
        fold_first(l1w1, 8), tile_b(l1b1), bd(l1w2), tile_b(l1b2),
        gw1, gb1.reshape(1, -1), gw2, gb2.reshape(1, -1),
        bd(l2w1[:16]), jnp.tile(l2w1[16:24], (1, _K)), tile_b(l2b1),
        bd(l2w2), tile_b(l2b2),
        g2w1, g2b1.reshape(1, -1), g2w2, g2b2.reshape(1, -1),
        bd(l3w1[:16]), jnp.tile(l3w1[16:24], (1, _K)), tile_b(l3b1),
        bd(l3w2), tile_b(l3b2),
        g3w1, g3b1.reshape(1, -1), g3w2, g3b2.reshape(1, -1),
        bd(l4w1[:16]), jnp.tile(l4w1[16:24], (1, _K)), fold_first(l4w1[24:27], 3),
        tile_b(l4b1), bd(l4w2), tile_b(l4b2),
    ]
    return consts


def _tc_forward(std, pos, nrm, g6, w, consts, blk0, nblk):
    nn = nblk * _T
    in_specs = [
        pl.BlockSpec(memory_space=pltpu.SMEM),
        pl.BlockSpec((_T, 3), lambda i: (blk0 + i, 0)),
        pl.BlockSpec((_T, 3), lambda i: (blk0 + i, 0)),
    ] + [pl.BlockSpec((_T, _K), lambda i: (i, 0)) for _ in range(6)] + [
        pl.BlockSpec((_T, _K), lambda i: (blk0 + i, 0)),
    ] + [
        pl.BlockSpec(c.shape, lambda i: tuple([0] * c.ndim)) for c in consts
    ]
    out = pl.pallas_call(
        _tc_body,
        grid=(nblk,),
        in_specs=in_specs,
        out_specs=pl.BlockSpec((_T, _K), lambda i: (i, 0)),
        out_shape=jax.ShapeDtypeStruct((nn, _K), jnp.float32),
        compiler_params=pltpu.CompilerParams(
            dimension_semantics=("parallel",)
        ),
    )(std, pos, nrm, *g6, w, *consts)
    return out


_S = 5


def kernel(pos, old_weights, normals, edge_index, dense_l, stddev, params):
    cols = edge_index[1]
    tables = [pos[:, 0], pos[:, 1], pos[:, 2],
              normals[:, 0], normals[:, 1], normals[:, 2]]
    w = old_weights.reshape(_N, _K)
    std = stddev.reshape(1, 1)
    consts = _make_consts(params)

    nn = _N // _S
    ne = _E // _S
    gathered = [_sc_gather6(tables, cols, c * ne, ne) for c in range(_S)]
    outs = []
    for c in range(_S):
        g6 = [a.reshape(nn, _K) for a in gathered[c]]
        outs.append(
            _tc_forward(std, pos, normals, g6, w, consts,
                        c * (nn // _T), nn // _T)
        )
    return jnp.concatenate(outs, axis=0)


# speedup vs baseline: 1.5038x; 1.0166x over previous
import functools

import jax
import jax.numpy as jnp
from jax import lax
from jax.experimental import pallas as pl
from jax.experimental.pallas import tpu as pltpu
from jax.experimental.pallas import tpu_sc as plsc

_N = 50000
_K = 16
_E = _N * _K
_T = 2000
_NC = 2
_NS = 16
_CH = 5000


def _sc_gather6(tables, cols, ebase, e):
    nw = _NC * _NS
    b_per_w = e // nw
    n_ch = b_per_w // _CH
    mesh = plsc.VectorSubcoreMesh(core_axis_name="c", subcore_axis_name="s")

    @functools.partial(
        pl.kernel,
        out_type=[jax.ShapeDtypeStruct((e,), jnp.float32) for _ in range(6)],
        mesh=mesh,
        scratch_types=[pltpu.VMEM((_CH,), jnp.int32)]
        + [pltpu.VMEM((_CH,), jnp.float32) for _ in range(6)]
        + [pltpu.SemaphoreType.DMA],
    )
    def gather_kernel(*refs):
        tbls = refs[0:6]
        idx_hbm = refs[6]
        outs = refs[7:13]
        idx_v = refs[13]
        vals = refs[14:20]
        sem = refs[20]
        wid = lax.axis_index("s") * _NC + lax.axis_index("c")
        base = wid * b_per_w

        @pl.loop(0, n_ch)
        def _(c):
            off = base + c * _CH
            pltpu.sync_copy(idx_hbm.at[pl.ds(ebase + off, _CH)], idx_v)
            copies = [
                pltpu.async_copy(tbls[j].at[idx_v], vals[j], sem)
                for j in range(6)
            ]
            for cp in copies:
                cp.wait()
            for j in range(6):
                pltpu.sync_copy(vals[j], outs[j].at[pl.ds(off, _CH)])

    return gather_kernel(*tables, cols)


def _tc_body(std_ref, pos_ref, nrm_ref, g0, g1, g2, g3, g4, g5, w_ref, *rest):
    (rmat,
     p1, b1t, bd12, b2t1,
     gw1, gb1, gw2, gb2,
     bda2, tb2, b1t2, bd22, b2t2,
     g2w1, g2b1, g2w2, g2b2,
     bda3, tb3, b1t3, bd23, b2t3,
     g3w1, g3b1, g3w2, g3b2,
     bd4a, tb4, pc4, b4t, bd4b, b4b,
     out_ref) = rest

    def mm(a, b):
        return jnp.dot(a, b[...], preferred_element_type=jnp.float32)

    s = 0.2 / std_ref[0, 0]
    prx, pry, prz = pos_ref[:, 0:1], pos_ref[:, 1:2], pos_ref[:, 2:3]
    nrx, nry, nrz = nrm_ref[:, 0:1], nrm_ref[:, 1:2], nrm_ref[:, 2:3]

    pcx, pcy, pcz = g0[...], g1[...], g2[...]
    ncx, ncy, ncz = g3[...], g4[...], g5[...]

    cx = (pcx - prx) * s
    cy = (pcy - pry) * s
    cz = (pcz - prz) * s

    def sqn(u0, u1, u2):
        return u0 * u0 + u1 * u1 + u2 * u2

    s1 = sqn(nry * cz - nrz * cy, nrz * cx - nrx * cz, nrx * cy - nry * cx)
    s2 = sqn(ncy * cz - ncz * cy, ncz * cx - ncx * cz, ncx * cy - ncy * cx)
    s3 = sqn(nry * ncz - nrz * ncy, nrz * ncx - nrx * ncz, nrx * ncy - nry * ncx)
    rt = jnp.sqrt(jnp.concatenate([s1, s2, s3, sqn(cx, cy, cz)], axis=1))
    dots = jnp.concatenate(
        [nrx * cx + nry * cy + nrz * cz,
         ncx * cx + ncy * cy + ncz * cz,
         nrx * ncx + nry * ncy + nrz * ncz], axis=1)
    ang = jnp.arctan2(rt[:, 0:48], dots)

    x128 = jnp.concatenate([cx, cy, cz, w_ref[...], rt[:, 48:64], ang], axis=1)
    h = jnp.maximum(mm(x128, p1) + b1t[...], 0.0)
    x16 = mm(h, bd12) + b2t1[...]

    gx = mm(x16, rmat)
    gin = jnp.concatenate([gx, nrm_ref[...]], axis=1)
    hg = jnp.maximum(mm(gin, gw1) + gb1[...], 0.0)
    xg = mm(hg, gw2) + gb2[...]

    h = jnp.maximum(mm(x16, bda2) + mm(xg, tb2) + b1t2[...], 0.0)
    x16 = mm(h, bd22) + b2t2[...]

    gx = mm(x16, rmat)
    hg = jnp.maximum(mm(gx, g2w1) + g2b1[...], 0.0)
    xg = mm(hg, g2w2) + g2b2[...]

    h = jnp.maximum(mm(x16, bda3) + mm(xg, tb3) + b1t3[...], 0.0)
    x16 = mm(h, bd23) + b2t3[...]

    gx = mm(x16, rmat)
    hg = jnp.maximum(mm(gx, g3w1) + g3b1[...], 0.0)
    xg = mm(hg, g3w2) + g3b2[...]

    qw, qx, qy, qz = xg[:, 0:1], xg[:, 1:2], xg[:, 2:3], xg[:, 3:4]
    d = qw * qw + qx * qx + qy * qy + qz * qz + 1e-16
    r = 1.0 / d
    m00 = d - 2 * (qy * qy + qz * qz)
    m01 = 2 * (qx * qy - qw * qz)
    m02 = 2 * (qx * qz + qw * qy)
    m10 = 2 * (qx * qy + qw * qz)
    m11 = d - 2 * (qx * qx + qz * qz)
    m12 = 2 * (qy * qz - qw * qx)
    m20 = 2 * (qx * qz - qw * qy)
    m21 = 2 * (qy * qz + qw * qx)
    m22 = d - 2 * (qx * qx + qy * qy)
    rcx = (m00 * cx + m01 * cy + m02 * cz) * r
    rcy = (m10 * cx + m11 * cy + m12 * cz) * r
    rcz = (m20 * cx + m21 * cy + m22 * cz) * r
    rc = jnp.concatenate([rcx, rcy, rcz], axis=1)

    h = jnp.maximum(
        mm(x16, bd4a) + mm(xg[:, 4:12], tb4) + mm(rc, pc4) + b4t[...], 0.0
    )
    y = mm(h, bd4b) + b4b[...]

    ymax = jnp.max(y, axis=1, keepdims=True)
    ey = jnp.exp(y - ymax)
    out_ref[...] = ey / jnp.sum(ey, axis=1, keepdims=True)


def _make_consts(params):
    eye = jnp.eye(_K, dtype=jnp.float32)

    def bd(w):
        return jnp.kron(eye, w)

    def fold_first(w, fin):
        return jnp.einsum("fh,kK->fkKh", w, eye).reshape(fin * _K, _K * w.shape[1])

    def tile_b(b):
        return jnp.tile(b.reshape(1, -1), (1, _K))

    l1w1, l1b1, l1w2, l1b2 = params["layer1"]
    gw1, gb1, gw2, gb2 = params["layerg"]
    l2w1, l2b1, l2w2, l2b2 = params["layer2"]
    g2w1, g2b1, g2w2, g2b2 = params["layerg2"]
    l3w1, l3b1, l3w2, l3b2 = params["layer3"]
    g3w1, g3b1, g3w2, g3b2 = params["layerg3"]
    l4w1, l4b1, l4w2, l4b2 = params["layer4"]

    consts = [
        jnp.tile(eye, (_K, 1)) / _K,
        fold_first(l1w1, 8), tile_b(l1b1), bd(l1w2), tile_b(l1b2),
        gw1, gb1.reshape(1, -1), gw2, gb2.reshape(1, -1),
        bd(l2w1[:16]), jnp.tile(l2w1[16:24], (1, _K)), tile_b(l2b1),
        bd(l2w2), tile_b(l2b2),
        g2w1, g2b1.reshape(1, -1), g2w2, g2b2.reshape(1, -1),
        bd(l3w1[:16]), jnp.tile(l3w1[16:24], (1, _K)), tile_b(l3b1),
        bd(l3w2), tile_b(l3b2),
        g3w1, g3b1.reshape(1, -1), g3w2, g3b2.reshape(1, -1),
        bd(l4w1[:16]), jnp.tile(l4w1[16:24], (1, _K)), fold_first(l4w1[24:27], 3),
        tile_b(l4b1), bd(l4w2), tile_b(l4b2),
    ]
    return consts


def _tc_forward(std, pos, nrm, g6, w, consts, blk0, nblk):
    nn = nblk * _T
    in_specs = [
        pl.BlockSpec(memory_space=pltpu.SMEM),
        pl.BlockSpec((_T, 3), lambda i: (blk0 + i, 0)),
        pl.BlockSpec((_T, 3), lambda i: (blk0 + i, 0)),
    ] + [pl.BlockSpec((_T, _K), lambda i: (i, 0)) for _ in range(6)] + [
        pl.BlockSpec((_T, _K), lambda i: (blk0 + i, 0)),
    ] + [
        pl.BlockSpec(c.shape, lambda i: tuple([0] * c.ndim)) for c in consts
    ]
    out = pl.pallas_call(
        _tc_body,
        grid=(nblk,),
        in_specs=in_specs,
        out_specs=pl.BlockSpec((_T, _K), lambda i: (i, 0)),
        out_shape=jax.ShapeDtypeStruct((nn, _K), jnp.float32),
        compiler_params=pltpu.CompilerParams(
            dimension_semantics=("parallel",)
        ),
    )(std, pos, nrm, *g6, w, *consts)
    return out


_S = 5


def kernel(pos, old_weights, normals, edge_index, dense_l, stddev, params):
    cols = edge_index[1]
    tables = [pos[:, 0], pos[:, 1], pos[:, 2],
              normals[:, 0], normals[:, 1], normals[:, 2]]
    w = old_weights.reshape(_N, _K)
    std = stddev.reshape(1, 1)
    consts = _make_consts(params)

    nn = _N // _S
    ne = _E // _S
    gathered = [_sc_gather6(tables, cols, c * ne, ne) for c in range(_S)]
    outs = []
    for c in range(_S):
        g6 = [a.reshape(nn, _K) for a in gathered[c]]
        outs.append(
            _tc_forward(std, pos, normals, g6, w, consts,
                        c * (nn // _T), nn // _T)
        )
    return jnp.concatenate(outs, axis=0)
